# baseline (speedup 1.0000x reference)
import functools

import jax
import jax.numpy as jnp
from jax.experimental import pallas as pl

B_TOTAL = 4096
TA = 16
TB = 512
K = 512


def _conv_feats_kernel(x_ref, w1_ref, b1_ref, w2_ref, b2_ref, feats_ref):
    t = x_ref.shape[0]
    x = x_ref[...]
    xp = jnp.pad(x, ((0, 0), (1, 1), (1, 1)))
    p1 = jnp.concatenate(
        [xp[:, ky:ky + 28, kx:kx + 28].reshape(t, 28, 28, 1)
         for ky in range(3) for kx in range(3)], axis=-1)
    p1 = p1.reshape(t * 784, 9)
    h1 = jnp.dot(p1, w1_ref[...], preferred_element_type=jnp.float32)
    h1 = jnp.maximum(h1 + b1_ref[...], 0.0)
    h1 = h1.reshape(t, 14, 2, 14, 2, 32)
    h1p = jnp.max(jnp.max(h1, axis=4), axis=2)
    h1pp = jnp.pad(h1p, ((0, 0), (1, 1), (1, 1), (0, 0)))
    p2 = jnp.concatenate(
        [h1pp[:, ky:ky + 14, kx:kx + 14, :]
         for ky in range(3) for kx in range(3)], axis=-1)
    p2 = p2.reshape(t * 196, 288)
    h2 = jnp.dot(p2, w2_ref[...], preferred_element_type=jnp.float32)
    h2 = jnp.maximum(h2 + b2_ref[...], 0.0)
    h2 = h2.reshape(t, 7, 2, 7, 2, 64)
    h2p = jnp.max(jnp.max(h2, axis=4), axis=2)
    feats_ref[...] = h2p.reshape(t, 7 * 7 * 64)


def _fc_vq_kernel(f_ref, fcw_ref, fcb_ref, cct_ref, fc_out_ref, prob_ref, w_ref):
    t = f_ref.shape[0]
    f = jnp.dot(f_ref[...], fcw_ref[...], preferred_element_type=jnp.float32)
    f = f + fcb_ref[...]
    fc_out_ref[...] = f
    cct = cct_ref[...]
    a2 = jnp.sum(f * f, axis=1, keepdims=True)
    b2 = jnp.sum(cct * cct, axis=0, keepdims=True)
    fg = jnp.dot(f, cct, preferred_element_type=jnp.float32)
    d2 = jnp.maximum(a2 + b2 - 2.0 * fg, 0.0)
    dist = jnp.sqrt(d2 + 1e-12)
    dmin = jnp.min(dist, axis=1, keepdims=True)
    iota = jax.lax.broadcasted_iota(jnp.int32, (t, K), 1)
    label = jnp.min(jnp.where(dist == dmin, iota, K), axis=1, keepdims=True)
    w_ref[...] = (iota == label).astype(jnp.float32)
    z = -dist
    z = z - jnp.max(z, axis=1, keepdims=True)
    e = jnp.exp(z)
    prob_ref[...] = e / jnp.sum(e, axis=1, keepdims=True)


@functools.partial(jax.jit, static_argnums=())
def kernel(x, conv1_w, conv1_b, conv2_w, conv2_b, fc_w, fc_b, cluster_centers):
    b = x.shape[0]
    x3 = x.reshape(b, 28, 28)
    w1r = conv1_w.transpose(2, 3, 1, 0).reshape(9, 32)
    w2r = conv2_w.transpose(2, 3, 1, 0).reshape(288, 64)
    fcw = fc_w.reshape(K, 64, 7, 7).transpose(2, 3, 1, 0).reshape(3136, K)
    b1 = conv1_b.reshape(1, 32)
    b2 = conv2_b.reshape(1, 64)
    fcb = fc_b.reshape(1, K)
    cct = cluster_centers.T

    feats = pl.pallas_call(
        _conv_feats_kernel,
        grid=(b // TA,),
        in_specs=[
            pl.BlockSpec((TA, 28, 28), lambda i: (i, 0, 0)),
            pl.BlockSpec((9, 32), lambda i: (0, 0)),
            pl.BlockSpec((1, 32), lambda i: (0, 0)),
            pl.BlockSpec((288, 64), lambda i: (0, 0)),
            pl.BlockSpec((1, 64), lambda i: (0, 0)),
        ],
        out_specs=pl.BlockSpec((TA, 3136), lambda i: (i, 0)),
        out_shape=jax.ShapeDtypeStruct((b, 3136), jnp.float32),
    )(x3, w1r, b1, w2r, b2)

    fc_out, prob, w = pl.pallas_call(
        _fc_vq_kernel,
        grid=(b // TB,),
        in_specs=[
            pl.BlockSpec((TB, 3136), lambda i: (i, 0)),
            pl.BlockSpec((3136, K), lambda i: (0, 0)),
            pl.BlockSpec((1, K), lambda i: (0, 0)),
            pl.BlockSpec((K, K), lambda i: (0, 0)),
        ],
        out_specs=[
            pl.BlockSpec((TB, K), lambda i: (i, 0)),
            pl.BlockSpec((TB, K), lambda i: (i, 0)),
            pl.BlockSpec((TB, K), lambda i: (i, 0)),
        ],
        out_shape=[
            jax.ShapeDtypeStruct((b, K), jnp.float32),
            jax.ShapeDtypeStruct((b, K), jnp.float32),
            jax.ShapeDtypeStruct((b, K), jnp.float32),
        ],
    )(feats, fcw, fcb, cct)
    return (fc_out, prob, w)


# speedup vs baseline: 4.8288x; 4.8288x over previous
import numpy as np

import jax
import jax.numpy as jnp
from jax.experimental import pallas as pl

TA = 32
TB = 512
K = 512


def _conv_feats_kernel(x_ref, w1t_ref, b1t_ref, w2_ref, b2_ref, out_ref):
    t = TA
    r1 = t * 28
    x = x_ref[...]
    xp = jnp.pad(x, ((1, 1), (0, 0)))
    yi = jax.lax.broadcasted_iota(jnp.int32, (r1, 1), 0) % 28
    s0 = jnp.where(yi == 0, 0.0, xp[0:r1])
    s1 = xp[1:r1 + 1]
    s2 = jnp.where(yi == 27, 0.0, xp[2:r1 + 2])
    p1 = jnp.concatenate([s0, s1, s2], axis=1)
    h1 = jnp.dot(p1, w1t_ref[...], preferred_element_type=jnp.float32)
    h1 = jnp.maximum(h1 + b1t_ref[...], 0.0)
    h1r = h1.reshape(t * 14, 1792)
    hpy = jnp.maximum(h1r[:, :896], h1r[:, 896:])
    hq = hpy.reshape(t * 7, 1792)
    p2pre = jnp.concatenate(
        [jnp.concatenate(
            [jnp.maximum(hq[:, s2 * 896 + (2 * u) * 32:s2 * 896 + (2 * u) * 32 + 32],
                         hq[:, s2 * 896 + (2 * u + 1) * 32:s2 * 896 + (2 * u + 1) * 32 + 32])
             for s2 in range(2)], axis=1)
         for u in range(14)], axis=0)
    tw2 = t * 7
    r2 = 14 * tw2
    p0 = ((tw2 + 8) // 8) * 8
    zpad = jnp.zeros((p0, 64), jnp.float32)
    ppad = jnp.concatenate([zpad, p2pre, zpad], axis=0)
    y2p = jax.lax.broadcasted_iota(jnp.int32, (r2, 1), 0) % 7
    slices = []
    for kx in range(3):
        base = p0 + (kx - 1) * tw2
        a = jnp.where(y2p == 0, 0.0, ppad[base - 1:base - 1 + r2, 32:64])
        bm = ppad[base:base + r2]
        c = jnp.where(y2p == 6, 0.0, ppad[base + 1:base + 1 + r2, 0:32])
        slices += [a, bm, c]
    p2 = jnp.concatenate(slices, axis=1)
    h2 = jnp.dot(p2, w2_ref[...], preferred_element_type=jnp.float32)
    h2 = jnp.maximum(h2 + b2_ref[...], 0.0)
    hyp = jnp.maximum(h2[:, :64], h2[:, 64:])
    h2p = jnp.concatenate(
        [jnp.maximum(hyp[(2 * u) * tw2:(2 * u + 1) * tw2],
                     hyp[(2 * u + 1) * tw2:(2 * u + 2) * tw2])
         for u in range(7)], axis=0)
    out_ref[...] = h2p.reshape(7, tw2, 64)


def _fc_vq_kernel(f_ref, fcw_ref, fcb_ref, cct_ref, fc_out_ref, prob_ref, w_ref):
    t = f_ref.shape[0]
    f = jnp.dot(f_ref[...], fcw_ref[...], preferred_element_type=jnp.float32)
    f = f + fcb_ref[...]
    fc_out_ref[...] = f
    cct = cct_ref[...]
    a2 = jnp.sum(f * f, axis=1, keepdims=True)
    b2 = jnp.sum(cct * cct, axis=0, keepdims=True)
    fg = jnp.dot(f, cct, preferred_element_type=jnp.float32)
    d2 = jnp.maximum(a2 + b2 - 2.0 * fg, 0.0)
    dist = jnp.sqrt(d2 + 1e-12)
    dmin = jnp.min(dist, axis=1, keepdims=True)
    iota = jax.lax.broadcasted_iota(jnp.int32, (t, K), 1)
    label = jnp.min(jnp.where(dist == dmin, iota, K), axis=1, keepdims=True)
    w_ref[...] = (iota == label).astype(jnp.float32)
    z = -dist
    z = z - jnp.max(z, axis=1, keepdims=True)
    e = jnp.exp(z)
    prob_ref[...] = e / jnp.sum(e, axis=1, keepdims=True)


def kernel(x, conv1_w, conv1_b, conv2_w, conv2_b, fc_w, fc_b, cluster_centers):
    b = x.shape[0]
    x2d = x.reshape(b * 28, 28)
    w1t = jnp.zeros((3, 28, 28, 32), jnp.float32)
    xs = np.arange(28)
    for ky in range(3):
        for kx in range(3):
            xv = xs - (kx - 1)
            valid = (xv >= 0) & (xv < 28)
            w1t = w1t.at[ky, xs[valid], xv[valid], :].set(conv1_w[:, 0, ky, kx])
    w1t = w1t.reshape(84, 896)
    b1t = jnp.tile(conv1_b, 28).reshape(1, 896)
    w2r = jnp.zeros((3, 4, 32, 2, 64), jnp.float32)
    for kx in range(3):
        for yy in range(4):
            for s2o in range(2):
                ky = yy - s2o
                if 0 <= ky < 3:
                    w2r = w2r.at[kx, yy, :, s2o, :].set(conv2_w[:, :, ky, kx].T)
    w2r = w2r.reshape(384, 128)
    b2 = jnp.tile(conv2_b, 2).reshape(1, 128)
    fcw = fc_w.reshape(K, 64, 7, 7).transpose(3, 2, 1, 0).reshape(3136, K)
    fcb = fc_b.reshape(1, K)
    cct = cluster_centers.T

    feats_alt = pl.pallas_call(
        _conv_feats_kernel,
        grid=(b // TA,),
        in_specs=[
            pl.BlockSpec((TA * 28, 28), lambda i: (i, 0)),
            pl.BlockSpec((84, 896), lambda i: (0, 0)),
            pl.BlockSpec((1, 896), lambda i: (0, 0)),
            pl.BlockSpec((384, 128), lambda i: (0, 0)),
            pl.BlockSpec((1, 128), lambda i: (0, 0)),
        ],
        out_specs=pl.BlockSpec((7, TA * 7, 64), lambda i: (0, i, 0)),
        out_shape=jax.ShapeDtypeStruct((7, b * 7, 64), jnp.float32),
    )(x2d, w1t, b1t, w2r, b2)

    feats = feats_alt.reshape(7, b, 7, 64).transpose(1, 0, 2, 3).reshape(b, 3136)

    fc_out, prob, w = pl.pallas_call(
        _fc_vq_kernel,
        grid=(b // TB,),
        in_specs=[
            pl.BlockSpec((TB, 3136), lambda i: (i, 0)),
            pl.BlockSpec((3136, K), lambda i: (0, 0)),
            pl.BlockSpec((1, K), lambda i: (0, 0)),
            pl.BlockSpec((K, K), lambda i: (0, 0)),
        ],
        out_specs=[
            pl.BlockSpec((TB, K), lambda i: (i, 0)),
            pl.BlockSpec((TB, K), lambda i: (i, 0)),
            pl.BlockSpec((TB, K), lambda i: (i, 0)),
        ],
        out_shape=[
            jax.ShapeDtypeStruct((b, K), jnp.float32),
            jax.ShapeDtypeStruct((b, K), jnp.float32),
            jax.ShapeDtypeStruct((b, K), jnp.float32),
        ],
    )(feats, fcw, fcb, cct)
    return (fc_out, prob, w)


# speedup vs baseline: 5.4047x; 1.1193x over previous
import numpy as np

import jax
import jax.numpy as jnp
from jax.experimental import pallas as pl

TA = 32
TB = 512
K = 512


def _conv_feats_kernel(x_ref, w1t_ref, b1t_ref, w2_ref, b2_ref, out_ref):
    t = TA
    r1 = t * 7
    x = x_ref[...]
    xp = jnp.pad(x, ((1, 1), (0, 0)))
    yi = jax.lax.broadcasted_iota(jnp.int32, (r1, 1), 0) % 7
    prev = jnp.where(yi == 0, 0.0, xp[0:r1, 84:112])
    cur = xp[1:r1 + 1]
    nxt = jnp.where(yi == 6, 0.0, xp[2:r1 + 2, 0:28])
    p1 = jnp.concatenate([prev, cur, nxt], axis=1)
    h1 = jnp.dot(p1, w1t_ref[...], preferred_element_type=jnp.float32)
    h1 = jnp.maximum(h1 + b1t_ref[...], 0.0)
    hq = jnp.concatenate(
        [jnp.maximum(h1[:, 0:896], h1[:, 896:1792]),
         jnp.maximum(h1[:, 1792:2688], h1[:, 2688:3584])], axis=1)
    p2pre = jnp.concatenate(
        [jnp.concatenate(
            [jnp.maximum(hq[:, s2 * 896 + (2 * u) * 32:s2 * 896 + (2 * u) * 32 + 32],
                         hq[:, s2 * 896 + (2 * u + 1) * 32:s2 * 896 + (2 * u + 1) * 32 + 32])
             for s2 in range(2)], axis=1)
         for u in range(14)], axis=0)
    tw2 = t * 7
    r2 = 14 * tw2
    p0 = ((tw2 + 8) // 8) * 8
    zpad = jnp.zeros((p0, 64), jnp.float32)
    ppad = jnp.concatenate([zpad, p2pre, zpad], axis=0)
    y2p = jax.lax.broadcasted_iota(jnp.int32, (r2, 1), 0) % 7
    slices = []
    for kx in range(3):
        base = p0 + (kx - 1) * tw2
        a = jnp.where(y2p == 0, 0.0, ppad[base - 1:base - 1 + r2, 32:64])
        bm = ppad[base:base + r2]
        c = jnp.where(y2p == 6, 0.0, ppad[base + 1:base + 1 + r2, 0:32])
        slices += [a, bm, c]
    p2 = jnp.concatenate(slices, axis=1)
    h2 = jnp.dot(p2, w2_ref[...], preferred_element_type=jnp.float32)
    h2 = jnp.maximum(h2 + b2_ref[...], 0.0)
    hyp = jnp.maximum(h2[:, :64], h2[:, 64:])
    h2p = jnp.concatenate(
        [jnp.maximum(hyp[(2 * u) * tw2:(2 * u + 1) * tw2],
                     hyp[(2 * u + 1) * tw2:(2 * u + 2) * tw2])
         for u in range(7)], axis=0)
    out_ref[...] = h2p.reshape(7, tw2, 64)


def _fc_vq_kernel(f_ref, fcw_ref, fcb_ref, cct_ref, fc_out_ref, prob_ref, w_ref):
    t = f_ref.shape[0]
    f = jnp.dot(f_ref[...], fcw_ref[...], preferred_element_type=jnp.float32)
    f = f + fcb_ref[...]
    fc_out_ref[...] = f
    cct = cct_ref[...]
    a2 = jnp.sum(f * f, axis=1, keepdims=True)
    b2 = jnp.sum(cct * cct, axis=0, keepdims=True)
    fg = jnp.dot(f, cct, preferred_element_type=jnp.float32)
    d2 = jnp.maximum(a2 + b2 - 2.0 * fg, 0.0)
    dist = jnp.sqrt(d2 + 1e-12)
    dmin = jnp.min(dist, axis=1, keepdims=True)
    iota = jax.lax.broadcasted_iota(jnp.int32, (t, K), 1)
    label = jnp.min(jnp.where(dist == dmin, iota, K), axis=1, keepdims=True)
    w_ref[...] = (iota == label).astype(jnp.float32)
    z = -dist
    z = z - jnp.max(z, axis=1, keepdims=True)
    e = jnp.exp(z)
    prob_ref[...] = e / jnp.sum(e, axis=1, keepdims=True)


def kernel(x, conv1_w, conv1_b, conv2_w, conv2_b, fc_w, fc_b, cluster_centers):
    b = x.shape[0]
    x2d = x.reshape(b * 7, 112)
    w1t = jnp.zeros((6, 28, 2, 2, 28, 32), jnp.float32)
    xs = np.arange(28)
    for yy in range(6):
        for s2 in range(2):
            for s in range(2):
                ky = yy - 2 * s2 - s
                if not 0 <= ky < 3:
                    continue
                for kx in range(3):
                    xv = xs - (kx - 1)
                    valid = (xv >= 0) & (xv < 28)
                    w1t = w1t.at[yy, xs[valid], s2, s, xv[valid], :].set(
                        conv1_w[:, 0, ky, kx])
    w1t = w1t.reshape(168, 3584)
    b1t = jnp.tile(conv1_b, 112).reshape(1, 3584)
    w2r = jnp.zeros((3, 4, 32, 2, 64), jnp.float32)
    for kx in range(3):
        for yy in range(4):
            for s2o in range(2):
                ky = yy - s2o
                if 0 <= ky < 3:
                    w2r = w2r.at[kx, yy, :, s2o, :].set(conv2_w[:, :, ky, kx].T)
    w2r = w2r.reshape(384, 128)
    b2 = jnp.tile(conv2_b, 2).reshape(1, 128)
    fcw = fc_w.reshape(K, 64, 7, 7).transpose(3, 2, 1, 0).reshape(3136, K)
    fcb = fc_b.reshape(1, K)
    cct = cluster_centers.T

    feats_alt = pl.pallas_call(
        _conv_feats_kernel,
        grid=(b // TA,),
        in_specs=[
            pl.BlockSpec((TA * 7, 112), lambda i: (i, 0)),
            pl.BlockSpec((168, 3584), lambda i: (0, 0)),
            pl.BlockSpec((1, 3584), lambda i: (0, 0)),
            pl.BlockSpec((384, 128), lambda i: (0, 0)),
            pl.BlockSpec((1, 128), lambda i: (0, 0)),
        ],
        out_specs=pl.BlockSpec((7, TA * 7, 64), lambda i: (0, i, 0)),
        out_shape=jax.ShapeDtypeStruct((7, b * 7, 64), jnp.float32),
    )(x2d, w1t, b1t, w2r, b2)

    feats = feats_alt.reshape(7, b, 7, 64).transpose(1, 0, 2, 3).reshape(b, 3136)

    fc_out, prob, w = pl.pallas_call(
        _fc_vq_kernel,
        grid=(b // TB,),
        in_specs=[
            pl.BlockSpec((TB, 3136), lambda i: (i, 0)),
            pl.BlockSpec((3136, K), lambda i: (0, 0)),
            pl.BlockSpec((1, K), lambda i: (0, 0)),
            pl.BlockSpec((K, K), lambda i: (0, 0)),
        ],
        out_specs=[
            pl.BlockSpec((TB, K), lambda i: (i, 0)),
            pl.BlockSpec((TB, K), lambda i: (i, 0)),
            pl.BlockSpec((TB, K), lambda i: (i, 0)),
        ],
        out_shape=[
            jax.ShapeDtypeStruct((b, K), jnp.float32),
            jax.ShapeDtypeStruct((b, K), jnp.float32),
            jax.ShapeDtypeStruct((b, K), jnp.float32),
        ],
    )(feats, fcw, fcb, cct)
    return (fc_out, prob, w)


# speedup vs baseline: 6.5347x; 1.2091x over previous
import numpy as np

import jax
import jax.numpy as jnp
from jax.experimental import pallas as pl

TA = 64
TB = 512
K = 512


def _conv_feats_kernel(x_ref, w1t_ref, b1t_ref, w2_ref, b2_ref, out_ref):
    t = TA
    r1 = t * 7
    x = x_ref[...]
    xp = jnp.pad(x, ((1, 1), (0, 0)))
    yi = jax.lax.broadcasted_iota(jnp.int32, (r1, 1), 0) % 7
    prev = jnp.where(yi == 0, 0.0, xp[0:r1, 84:112])
    cur = xp[1:r1 + 1]
    nxt = jnp.where(yi == 6, 0.0, xp[2:r1 + 2, 0:28])
    p1 = jnp.concatenate([prev, cur, nxt], axis=1)
    h1 = jnp.dot(p1, w1t_ref[...], preferred_element_type=jnp.float32)
    h1 = jnp.maximum(h1 + b1t_ref[...], 0.0)
    hq = jnp.concatenate(
        [jnp.maximum(h1[:, 0:896], h1[:, 896:1792]),
         jnp.maximum(h1[:, 1792:2688], h1[:, 2688:3584])], axis=1)
    p2pre = jnp.concatenate(
        [jnp.concatenate(
            [jnp.maximum(hq[:, s2 * 896 + (2 * u) * 32:s2 * 896 + (2 * u) * 32 + 32],
                         hq[:, s2 * 896 + (2 * u + 1) * 32:s2 * 896 + (2 * u + 1) * 32 + 32])
             for s2 in range(2)], axis=1)
         for u in range(14)], axis=0)
    tw2 = t * 7
    r2 = 14 * tw2
    p0 = ((tw2 + 8) // 8) * 8
    zpad = jnp.zeros((p0, 64), jnp.float32)
    ppad = jnp.concatenate([zpad, p2pre, zpad], axis=0)
    y2p = jax.lax.broadcasted_iota(jnp.int32, (r2, 1), 0) % 7
    slices = []
    for kx in range(3):
        base = p0 + (kx - 1) * tw2
        a = jnp.where(y2p == 0, 0.0, ppad[base - 1:base - 1 + r2, 32:64])
        bm = ppad[base:base + r2]
        c = jnp.where(y2p == 6, 0.0, ppad[base + 1:base + 1 + r2, 0:32])
        slices += [a, bm, c]
    p2 = jnp.concatenate(slices, axis=1)
    h2 = jnp.dot(p2, w2_ref[...], preferred_element_type=jnp.float32)
    h2 = jnp.maximum(h2 + b2_ref[...], 0.0)
    hyp = jnp.maximum(h2[:, :64], h2[:, 64:])
    h2p = jnp.concatenate(
        [jnp.maximum(hyp[(2 * u) * tw2:(2 * u + 1) * tw2],
                     hyp[(2 * u + 1) * tw2:(2 * u + 2) * tw2])
         for u in range(7)] + [jnp.zeros((tw2, 64), jnp.float32)],
        axis=1)
    out_ref[...] = h2p.reshape(t, 3584)


def _fc_vq_kernel(f_ref, fcw_ref, fcb_ref, cct_ref, fc_out_ref, prob_ref, w_ref):
    t = f_ref.shape[0]
    f = jnp.dot(f_ref[...], fcw_ref[...], preferred_element_type=jnp.float32)
    f = f + fcb_ref[...]
    fc_out_ref[...] = f
    cct = cct_ref[...]
    a2 = jnp.sum(f * f, axis=1, keepdims=True)
    b2 = jnp.sum(cct * cct, axis=0, keepdims=True)
    fg = jnp.dot(f, cct, preferred_element_type=jnp.float32)
    d2 = jnp.maximum(a2 + b2 - 2.0 * fg, 0.0)
    dist = jnp.sqrt(d2 + 1e-12)
    dmin = jnp.min(dist, axis=1, keepdims=True)
    iota = jax.lax.broadcasted_iota(jnp.int32, (t, K), 1)
    label = jnp.min(jnp.where(dist == dmin, iota, K), axis=1, keepdims=True)
    w_ref[...] = (iota == label).astype(jnp.float32)
    z = -dist
    z = z - jnp.max(z, axis=1, keepdims=True)
    e = jnp.exp(z)
    prob_ref[...] = e / jnp.sum(e, axis=1, keepdims=True)


def kernel(x, conv1_w, conv1_b, conv2_w, conv2_b, fc_w, fc_b, cluster_centers):
    b = x.shape[0]
    x2d = x.reshape(b * 7, 112)
    w1t = jnp.zeros((6, 28, 2, 2, 28, 32), jnp.float32)
    xs = np.arange(28)
    for yy in range(6):
        for s2 in range(2):
            for s in range(2):
                ky = yy - 2 * s2 - s
                if not 0 <= ky < 3:
                    continue
                for kx in range(3):
                    xv = xs - (kx - 1)
                    valid = (xv >= 0) & (xv < 28)
                    w1t = w1t.at[yy, xs[valid], s2, s, xv[valid], :].set(
                        conv1_w[:, 0, ky, kx])
    w1t = w1t.reshape(168, 3584)
    b1t = jnp.tile(conv1_b, 112).reshape(1, 3584)
    w2r = jnp.zeros((3, 4, 32, 2, 64), jnp.float32)
    for kx in range(3):
        for yy in range(4):
            for s2o in range(2):
                ky = yy - s2o
                if 0 <= ky < 3:
                    w2r = w2r.at[kx, yy, :, s2o, :].set(conv2_w[:, :, ky, kx].T)
    w2r = w2r.reshape(384, 128)
    b2 = jnp.tile(conv2_b, 2).reshape(1, 128)
    fcw = jnp.zeros((7, 8, 64, K), jnp.float32)
    fcw = fcw.at[:, :7].set(fc_w.reshape(K, 64, 7, 7).transpose(2, 3, 1, 0))
    fcw = fcw.reshape(3584, K)
    fcb = fc_b.reshape(1, K)
    cct = cluster_centers.T

    feats_alt = pl.pallas_call(
        _conv_feats_kernel,
        grid=(b // TA,),
        in_specs=[
            pl.BlockSpec((TA * 7, 112), lambda i: (i, 0)),
            pl.BlockSpec((168, 3584), lambda i: (0, 0)),
            pl.BlockSpec((1, 3584), lambda i: (0, 0)),
            pl.BlockSpec((384, 128), lambda i: (0, 0)),
            pl.BlockSpec((1, 128), lambda i: (0, 0)),
        ],
        out_specs=pl.BlockSpec((TA, 3584), lambda i: (i, 0)),
        out_shape=jax.ShapeDtypeStruct((b, 3584), jnp.float32),
    )(x2d, w1t, b1t, w2r, b2)

    fc_out, prob, w = pl.pallas_call(
        _fc_vq_kernel,
        grid=(b // TB,),
        in_specs=[
            pl.BlockSpec((TB, 3584), lambda i: (i, 0)),
            pl.BlockSpec((3584, K), lambda i: (0, 0)),
            pl.BlockSpec((1, K), lambda i: (0, 0)),
            pl.BlockSpec((K, K), lambda i: (0, 0)),
        ],
        out_specs=[
            pl.BlockSpec((TB, K), lambda i: (i, 0)),
            pl.BlockSpec((TB, K), lambda i: (i, 0)),
            pl.BlockSpec((TB, K), lambda i: (i, 0)),
        ],
        out_shape=[
            jax.ShapeDtypeStruct((b, K), jnp.float32),
            jax.ShapeDtypeStruct((b, K), jnp.float32),
            jax.ShapeDtypeStruct((b, K), jnp.float32),
        ],
    )(feats_alt, fcw, fcb, cct)
    return (fc_out, prob, w)
